# baseline (speedup 1.0000x reference)
import jax
import jax.numpy as jnp
from jax import lax
from jax.experimental import pallas as pl
from jax.experimental.pallas import tpu as pltpu
from jax.experimental.pallas import tpu_sc as plsc

N_NODES = 100000
HIDDEN = 128
N_PAD = 102400
CHUNK = 128
K0 = 18
K1 = 32
KMAX = max(K0, K1)


def _body(in_idx, out_idx, w_in, w_out, out, idx_a, idx_b, ba0, ba1, bb0, bb1,
          sa0, sa1, sb0, sb1):
  cid = lax.axis_index("c")
  sid = lax.axis_index("s")

  bufs = ((ba0, bb0, sa0, sb0), (ba1, bb1, sa1, sb1))

  def run(n_chunks, chunk0):
    nrows = n_chunks * CHUNK
    pltpu.sync_copy(in_idx.at[pl.ds(chunk0 * CHUNK, nrows)],
                    idx_a.at[pl.ds(0, nrows)])
    pltpu.sync_copy(out_idx.at[pl.ds(chunk0 * CHUNK, nrows)],
                    idx_b.at[pl.ds(0, nrows)])

    def issue(j, slot):
      ba, bb, sa, sb = bufs[slot]
      ia = idx_a.at[pl.ds(j * CHUNK, CHUNK)]
      ib = idx_b.at[pl.ds(j * CHUNK, CHUNK)]
      pltpu.async_copy(w_in.at[ia], ba, sa)
      pltpu.async_copy(w_out.at[ib], bb, sb)

    def finish(j, slot):
      ba, bb, sa, sb = bufs[slot]
      ia = idx_a.at[pl.ds(j * CHUNK, CHUNK)]
      ib = idx_b.at[pl.ds(j * CHUNK, CHUNK)]
      pltpu.make_async_copy(w_in.at[ia], ba, sa).wait()
      pltpu.make_async_copy(w_out.at[ib], bb, sb).wait()

      @plsc.parallel_loop(0, CHUNK, unroll=4)
      def _(r):
        for k in range(HIDDEN // 16):
          s = pl.ds(k * 16, 16)
          plsc.addupdate(ba.at[r, s], bb[r, s])

      pltpu.sync_copy(ba, out.at[pl.ds((chunk0 + j) * CHUNK, CHUNK)])

    issue(0, 0)

    def pair_body(p, carry):
      for s in range(2):
        j = 2 * p + s

        @pl.when(j + 1 < n_chunks)
        def _():
          issue(j + 1, 1 - s)

        finish(j, s)
      return carry

    lax.fori_loop(0, n_chunks // 2, pair_body, 0)

  @pl.when(cid == 0)
  def _():
    run(K0, sid * K0)

  @pl.when(cid == 1)
  def _():
    run(K1, 16 * K0 + sid * K1)


@jax.jit
def kernel(in_deg, out_deg, W_in, W_out):
  pad = N_PAD - N_NODES
  in_p = jnp.pad(in_deg.astype(jnp.int32), (0, pad))
  out_p = jnp.pad(out_deg.astype(jnp.int32), (0, pad))

  mesh = plsc.VectorSubcoreMesh(core_axis_name="c", subcore_axis_name="s")
  f = pl.kernel(
      _body,
      out_type=jax.ShapeDtypeStruct((N_PAD, HIDDEN), jnp.float32),
      mesh=mesh,
      scratch_types=[
          pltpu.VMEM((KMAX * CHUNK,), jnp.int32),
          pltpu.VMEM((KMAX * CHUNK,), jnp.int32),
          pltpu.VMEM((CHUNK, HIDDEN), jnp.float32),
          pltpu.VMEM((CHUNK, HIDDEN), jnp.float32),
          pltpu.VMEM((CHUNK, HIDDEN), jnp.float32),
          pltpu.VMEM((CHUNK, HIDDEN), jnp.float32),
          pltpu.SemaphoreType.DMA,
          pltpu.SemaphoreType.DMA,
          pltpu.SemaphoreType.DMA,
          pltpu.SemaphoreType.DMA,
      ],
  )
  res = f(in_p, out_p, W_in, W_out)
  return res[:N_NODES]


# speedup vs baseline: 1.0315x; 1.0315x over previous
import jax
import jax.numpy as jnp
from jax import lax
from jax.experimental import pallas as pl
from jax.experimental.pallas import tpu as pltpu
from jax.experimental.pallas import tpu_sc as plsc

N_NODES = 100000
HIDDEN = 128
N_PAD = 102400
CHUNK = 128
K0 = 32
K1 = 18
KMAX = max(K0, K1)


def _body(in_idx, out_idx, w_in, w_out, out, idx_a, idx_b, ba0, ba1, bb0, bb1,
          sa0, sa1, sb0, sb1):
  cid = lax.axis_index("c")
  sid = lax.axis_index("s")

  bufs = ((ba0, bb0, sa0, sb0), (ba1, bb1, sa1, sb1))

  def run(n_chunks, chunk0):
    nrows = n_chunks * CHUNK
    pltpu.sync_copy(in_idx.at[pl.ds(chunk0 * CHUNK, nrows)],
                    idx_a.at[pl.ds(0, nrows)])
    pltpu.sync_copy(out_idx.at[pl.ds(chunk0 * CHUNK, nrows)],
                    idx_b.at[pl.ds(0, nrows)])

    def issue(j, slot):
      ba, bb, sa, sb = bufs[slot]
      ia = idx_a.at[pl.ds(j * CHUNK, CHUNK)]
      ib = idx_b.at[pl.ds(j * CHUNK, CHUNK)]
      pltpu.async_copy(w_in.at[ia], ba, sa)
      pltpu.async_copy(w_out.at[ib], bb, sb)

    def finish(j, slot):
      ba, bb, sa, sb = bufs[slot]
      ia = idx_a.at[pl.ds(j * CHUNK, CHUNK)]
      ib = idx_b.at[pl.ds(j * CHUNK, CHUNK)]
      pltpu.make_async_copy(w_in.at[ia], ba, sa).wait()
      pltpu.make_async_copy(w_out.at[ib], bb, sb).wait()

      @plsc.parallel_loop(0, CHUNK, unroll=4)
      def _(r):
        for k in range(HIDDEN // 16):
          s = pl.ds(k * 16, 16)
          plsc.addupdate(ba.at[r, s], bb[r, s])

      pltpu.sync_copy(ba, out.at[pl.ds((chunk0 + j) * CHUNK, CHUNK)])

    issue(0, 0)

    def pair_body(p, carry):
      for s in range(2):
        j = 2 * p + s

        @pl.when(j + 1 < n_chunks)
        def _():
          issue(j + 1, 1 - s)

        finish(j, s)
      return carry

    lax.fori_loop(0, n_chunks // 2, pair_body, 0)

  @pl.when(cid == 0)
  def _():
    run(K0, sid * K0)

  @pl.when(cid == 1)
  def _():
    run(K1, 16 * K0 + sid * K1)


@jax.jit
def kernel(in_deg, out_deg, W_in, W_out):
  pad = N_PAD - N_NODES
  in_p = jnp.pad(in_deg.astype(jnp.int32), (0, pad))
  out_p = jnp.pad(out_deg.astype(jnp.int32), (0, pad))

  mesh = plsc.VectorSubcoreMesh(core_axis_name="c", subcore_axis_name="s")
  f = pl.kernel(
      _body,
      out_type=jax.ShapeDtypeStruct((N_PAD, HIDDEN), jnp.float32),
      mesh=mesh,
      scratch_types=[
          pltpu.VMEM((KMAX * CHUNK,), jnp.int32),
          pltpu.VMEM((KMAX * CHUNK,), jnp.int32),
          pltpu.VMEM((CHUNK, HIDDEN), jnp.float32),
          pltpu.VMEM((CHUNK, HIDDEN), jnp.float32),
          pltpu.VMEM((CHUNK, HIDDEN), jnp.float32),
          pltpu.VMEM((CHUNK, HIDDEN), jnp.float32),
          pltpu.SemaphoreType.DMA,
          pltpu.SemaphoreType.DMA,
          pltpu.SemaphoreType.DMA,
          pltpu.SemaphoreType.DMA,
      ],
  )
  res = f(in_p, out_p, W_in, W_out)
  return res[:N_NODES]


# speedup vs baseline: 2.5043x; 2.4278x over previous
import jax
import jax.numpy as jnp
from jax import lax
from jax.experimental import pallas as pl
from jax.experimental.pallas import tpu as pltpu
from jax.experimental.pallas import tpu_sc as plsc

N_NODES = 100000
HIDDEN = 128
N_PAD = 102400
CHUNK = 128
PER_W = N_PAD // 32
N_CHUNKS = PER_W // CHUNK
VOCAB2 = 1024


def _body(in_idx, out_idx, w2, out, idx_a, idx_b, ba0, ba1, bb0, bb1, spm,
          sa0, sa1, sb0, sb1, sst):
  cid = lax.axis_index("c")
  sid = lax.axis_index("s")
  wid = sid * 2 + cid

  rows_per_tile = VOCAB2 // 16
  pltpu.async_copy(w2.at[pl.ds(sid * rows_per_tile, rows_per_tile)],
                   spm.at[pl.ds(sid * rows_per_tile, rows_per_tile)],
                   sst).wait()
  plsc.subcore_barrier()

  pltpu.sync_copy(in_idx.at[pl.ds(wid * PER_W, PER_W)], idx_a)
  pltpu.sync_copy(out_idx.at[pl.ds(wid * PER_W, PER_W)], idx_b)

  bufs = ((ba0, bb0, sa0, sb0), (ba1, bb1, sa1, sb1))

  def issue(j, slot):
    ba, bb, sa, sb = bufs[slot]
    ia = idx_a.at[pl.ds(j * CHUNK, CHUNK)]
    ib = idx_b.at[pl.ds(j * CHUNK, CHUNK)]
    pltpu.async_copy(spm.at[ia], ba, sa)
    pltpu.async_copy(spm.at[ib], bb, sb)

  def finish(j, slot):
    ba, bb, sa, sb = bufs[slot]
    ia = idx_a.at[pl.ds(j * CHUNK, CHUNK)]
    ib = idx_b.at[pl.ds(j * CHUNK, CHUNK)]
    pltpu.make_async_copy(spm.at[ia], ba, sa).wait()
    pltpu.make_async_copy(spm.at[ib], bb, sb).wait()

    @plsc.parallel_loop(0, CHUNK, unroll=4)
    def _(r):
      for k in range(HIDDEN // 16):
        s = pl.ds(k * 16, 16)
        plsc.addupdate(ba.at[r, s], bb[r, s])

    pltpu.sync_copy(ba, out.at[pl.ds(wid * PER_W + j * CHUNK, CHUNK)])

  issue(0, 0)

  def pair_body(p, carry):
    for s in range(2):
      j = 2 * p + s

      @pl.when(j + 1 < N_CHUNKS)
      def _():
        issue(j + 1, 1 - s)

      finish(j, s)
    return carry

  lax.fori_loop(0, N_CHUNKS // 2, pair_body, 0)
  finish(N_CHUNKS - 1, (N_CHUNKS - 1) % 2)


@jax.jit
def kernel(in_deg, out_deg, W_in, W_out):
  pad = N_PAD - N_NODES
  in_p = jnp.pad(in_deg.astype(jnp.int32), (0, pad))
  out_p = jnp.pad(out_deg.astype(jnp.int32), (0, pad)) + 512
  w2 = jnp.concatenate([W_in, W_out], axis=0)

  mesh = plsc.VectorSubcoreMesh(core_axis_name="c", subcore_axis_name="s")
  f = pl.kernel(
      _body,
      out_type=jax.ShapeDtypeStruct((N_PAD, HIDDEN), jnp.float32),
      mesh=mesh,
      scratch_types=[
          pltpu.VMEM((PER_W,), jnp.int32),
          pltpu.VMEM((PER_W,), jnp.int32),
          pltpu.VMEM((CHUNK, HIDDEN), jnp.float32),
          pltpu.VMEM((CHUNK, HIDDEN), jnp.float32),
          pltpu.VMEM((CHUNK, HIDDEN), jnp.float32),
          pltpu.VMEM((CHUNK, HIDDEN), jnp.float32),
          pltpu.VMEM_SHARED((VOCAB2, HIDDEN), jnp.float32),
          pltpu.SemaphoreType.DMA,
          pltpu.SemaphoreType.DMA,
          pltpu.SemaphoreType.DMA,
          pltpu.SemaphoreType.DMA,
          pltpu.SemaphoreType.DMA,
      ],
  )
  res = f(in_p, out_p, w2)
  return res[:N_NODES]


# speedup vs baseline: 3.5905x; 1.4337x over previous
import jax
import jax.numpy as jnp
from jax import lax
from jax.experimental import pallas as pl
from jax.experimental.pallas import tpu as pltpu
from jax.experimental.pallas import tpu_sc as plsc

N_NODES = 100000
HIDDEN = 128
CHUNK = 128
N_CHUNKS = (N_NODES + CHUNK - 1) // CHUNK
N_IDX_PAD = N_CHUNKS * CHUNK
TAIL = N_NODES - (N_CHUNKS - 1) * CHUNK
BIG_W = N_CHUNKS - 24 * 32
KBIG, KSML = 25, 24
VOCAB2 = 1024


def _body(in_idx, out_idx, w2, out, idx_a, idx_b, ba0, ba1, bb0, bb1, spm,
          sa0, sa1, sb0, sb1, sst):
  cid = lax.axis_index("c")
  sid = lax.axis_index("s")
  wid = sid * 2 + cid

  rpt = VOCAB2 // 16
  pltpu.async_copy(w2.at[pl.ds(sid * rpt, rpt)], spm.at[pl.ds(sid * rpt, rpt)],
                   sst).wait()
  plsc.subcore_barrier()

  bufs = ((ba0, bb0, sa0, sb0), (ba1, bb1, sa1, sb1))

  def run(n_chunks, chunk0):
    nrows = n_chunks * CHUNK
    pltpu.sync_copy(in_idx.at[pl.ds(chunk0 * CHUNK, nrows)],
                    idx_a.at[pl.ds(0, nrows)])
    pltpu.sync_copy(out_idx.at[pl.ds(chunk0 * CHUNK, nrows)],
                    idx_b.at[pl.ds(0, nrows)])

    def issue(j, slot):
      ba, bb, sa, sb = bufs[slot]
      ia = idx_a.at[pl.ds(j * CHUNK, CHUNK)]
      ib = idx_b.at[pl.ds(j * CHUNK, CHUNK)]
      pltpu.async_copy(spm.at[ia], ba, sa)
      pltpu.async_copy(spm.at[ib], bb, sb)

    def finish(j, slot):
      ba, bb, sa, sb = bufs[slot]
      ia = idx_a.at[pl.ds(j * CHUNK, CHUNK)]
      ib = idx_b.at[pl.ds(j * CHUNK, CHUNK)]
      pltpu.make_async_copy(spm.at[ia], ba, sa).wait()
      pltpu.make_async_copy(spm.at[ib], bb, sb).wait()

      @plsc.parallel_loop(0, CHUNK, unroll=4)
      def _(r):
        for k in range(HIDDEN // 16):
          s = pl.ds(k * 16, 16)
          plsc.addupdate(ba.at[r, s], bb[r, s])

      g = chunk0 + j

      @pl.when(g < N_CHUNKS - 1)
      def _():
        pltpu.sync_copy(ba, out.at[pl.ds(g * CHUNK, CHUNK)])

      @pl.when(g == N_CHUNKS - 1)
      def _():
        pltpu.sync_copy(ba.at[pl.ds(0, TAIL)],
                        out.at[pl.ds(g * CHUNK, TAIL)])

    issue(0, 0)

    def pair_body(p, carry):
      for s in range(2):
        j = 2 * p + s

        @pl.when(j + 1 < n_chunks)
        def _():
          issue(j + 1, 1 - s)

        finish(j, s)
      return carry

    lax.fori_loop(0, n_chunks // 2, pair_body, 0)
    if n_chunks % 2:
      finish(n_chunks - 1, (n_chunks - 1) % 2)

  @pl.when(wid < BIG_W)
  def _():
    run(KBIG, wid * KBIG)

  @pl.when(wid >= BIG_W)
  def _():
    run(KSML, BIG_W * KBIG + (wid - BIG_W) * KSML)


@jax.jit
def kernel(in_deg, out_deg, W_in, W_out):
  pad = N_IDX_PAD - N_NODES
  in_p = jnp.pad(in_deg.astype(jnp.int32), (0, pad))
  out_p = jnp.pad(out_deg.astype(jnp.int32), (0, pad)) + 512
  w2 = jnp.concatenate([W_in, W_out], axis=0)

  mesh = plsc.VectorSubcoreMesh(core_axis_name="c", subcore_axis_name="s")
  f = pl.kernel(
      _body,
      out_type=jax.ShapeDtypeStruct((N_NODES, HIDDEN), jnp.float32),
      mesh=mesh,
      scratch_types=[
          pltpu.VMEM((KBIG * CHUNK,), jnp.int32),
          pltpu.VMEM((KBIG * CHUNK,), jnp.int32),
          pltpu.VMEM((CHUNK, HIDDEN), jnp.float32),
          pltpu.VMEM((CHUNK, HIDDEN), jnp.float32),
          pltpu.VMEM((CHUNK, HIDDEN), jnp.float32),
          pltpu.VMEM((CHUNK, HIDDEN), jnp.float32),
          pltpu.VMEM_SHARED((VOCAB2, HIDDEN), jnp.float32),
          pltpu.SemaphoreType.DMA,
          pltpu.SemaphoreType.DMA,
          pltpu.SemaphoreType.DMA,
          pltpu.SemaphoreType.DMA,
          pltpu.SemaphoreType.DMA,
      ],
  )
  return f(in_p, out_p, w2)
